# baseline (speedup 1.0000x reference)
import jax
import jax.numpy as jnp
from jax.experimental import pallas as pl

_SMOOTHING = 0.1
_CONFIDENCE = 1.0 - _SMOOTHING
_V = 100000
_B = 1024
_FILL = _SMOOTHING / (_V - 2)

_VR = 4000


def _smooth_block(ids_ref, out_ref):
    j = pl.program_id(0)
    ids = ids_ref[0, :]
    vocab = jax.lax.broadcasted_iota(jnp.int32, (_VR, _B), 0) + j * _VR
    val = jnp.where(vocab == ids[None, :], _CONFIDENCE, _FILL)
    out_ref[...] = jnp.where(vocab == 0, 0.0, val)


def kernel(tgt_tok_id):
    ids = tgt_tok_id.reshape(1, _B).astype(jnp.int32)
    out_t = pl.pallas_call(
        _smooth_block,
        grid=(_V // _VR,),
        in_specs=[pl.BlockSpec((1, _B), lambda j: (0, 0))],
        out_specs=pl.BlockSpec((_VR, _B), lambda j: (j, 0)),
        out_shape=jax.ShapeDtypeStruct((_V, _B), jnp.float32),
    )(ids)
    return jnp.transpose(out_t)


# speedup vs baseline: 1.0237x; 1.0237x over previous
import jax
import jax.numpy as jnp
from jax.experimental import pallas as pl

_SMOOTHING = 0.1
_CONFIDENCE = 1.0 - _SMOOTHING
_V = 100000
_B = 1024
_FILL = _SMOOTHING / (_V - 2)

_VR = 1000


def _smooth_block(ids_ref, out_ref):
    j = pl.program_id(0)
    ids = ids_ref[0, :]
    vocab = jax.lax.broadcasted_iota(jnp.int32, (_VR, _B), 0) + j * _VR
    val = jnp.where(vocab == ids[None, :], _CONFIDENCE, _FILL)
    out_ref[...] = jnp.where(vocab == 0, 0.0, val)


def kernel(tgt_tok_id):
    ids = tgt_tok_id.reshape(1, _B).astype(jnp.int32)
    out_t = pl.pallas_call(
        _smooth_block,
        grid=(_V // _VR,),
        in_specs=[pl.BlockSpec((1, _B), lambda j: (0, 0))],
        out_specs=pl.BlockSpec((_VR, _B), lambda j: (j, 0)),
        out_shape=jax.ShapeDtypeStruct((_V, _B), jnp.float32),
    )(ids)
    return jnp.transpose(out_t)
